# baseline (speedup 1.0000x reference)
import functools
import math

import jax
import jax.numpy as jnp
from jax import lax
from jax.experimental import pallas as pl
from jax.experimental.pallas import tpu as pltpu
from jax.experimental.pallas import tpu_sc as plsc

PRE_NMS_TOP_N = 2000
POST_NMS_TOP_N = 1000
NMS_THRESH = 0.7
MIN_SIZE = 1.0
IMG_H = 1024.0
IMG_W = 1024.0
BBOX_XFORM_CLIP = float(math.log(1000.0 / 16.0))

T = 2048
BLK = 128
NBLK = T // BLK
OUT_PAD = 1024


def _pair_iou_mask(x1a, y1a, x2a, y2a, ra, x1b, y1b, x2b, y2b, rb):
    ax1 = x1a[:, :, None]
    ay1 = y1a[:, :, None]
    ax2 = x2a[:, :, None]
    ay2 = y2a[:, :, None]
    bx1 = x1b[:, None, :]
    by1 = y1b[:, None, :]
    bx2 = x2b[:, None, :]
    by2 = y2b[:, None, :]
    iw = jnp.maximum(jnp.minimum(ax2, bx2) - jnp.maximum(ax1, bx1), 0.0)
    ih = jnp.maximum(jnp.minimum(ay2, by2) - jnp.maximum(ay1, by1), 0.0)
    inter = iw * ih
    iou = inter / (ra[:, :, None] + rb[:, None, :] - inter + 1e-9)
    return (iou > NMS_THRESH).astype(jnp.float32)


def _nms_kernel(scr_ref, d_ref, a_ref, out_ref, m_ref, keep_ref):
    B = scr_ref.shape[0]
    npad = T - scr_ref.shape[1]

    def _pad(v, const=0.0):
        return jnp.concatenate(
            [v, jnp.full((B, npad), const, jnp.float32)], axis=1)

    scr = _pad(scr_ref[...], -1e9)
    a1 = _pad(a_ref[:, 0, :])
    a2 = _pad(a_ref[:, 1, :])
    a3 = _pad(a_ref[:, 2, :])
    a4 = _pad(a_ref[:, 3, :])

    aw = a3 - a1
    ah = a4 - a2
    acx = a1 + 0.5 * aw
    acy = a2 + 0.5 * ah
    dw = jnp.minimum(_pad(d_ref[:, 2, :]), BBOX_XFORM_CLIP)
    dh = jnp.minimum(_pad(d_ref[:, 3, :]), BBOX_XFORM_CLIP)
    pcx = _pad(d_ref[:, 0, :]) * aw + acx
    pcy = _pad(d_ref[:, 1, :]) * ah + acy
    pw = jnp.exp(dw) * aw
    ph = jnp.exp(dh) * ah
    x1 = jnp.clip(pcx - 0.5 * pw, 0.0, IMG_W)
    y1 = jnp.clip(pcy - 0.5 * ph, 0.0, IMG_H)
    x2 = jnp.clip(pcx + 0.5 * pw, 0.0, IMG_W)
    y2 = jnp.clip(pcy + 0.5 * ph, 0.0, IMG_H)
    score = 1.0 / (1.0 + jnp.exp(-scr))
    valid = ((x2 - x1 >= MIN_SIZE) & (y2 - y1 >= MIN_SIZE) & (score > 0.0))
    keep_ref[...] = valid.astype(jnp.float32)
    area = (x2 - x1) * (y2 - y1)

    lane = lax.broadcasted_iota(jnp.int32, (B, BLK), 1)

    done = jnp.zeros((), jnp.bool_)
    for b in range(NBLK):
        sl = slice(b * BLK, (b + 1) * BLK)

        @pl.when(jnp.logical_not(done))
        def _():
            bx1, by1, bx2, by2, br = (x1[:, sl], y1[:, sl], x2[:, sl],
                                      y2[:, sl], area[:, sl])
            m_ref[...] = _pair_iou_mask(bx1, by1, bx2, by2, br,
                                        bx1, by1, bx2, by2, br)

            def body(j, kb):
                kj = jnp.sum(kb * (lane == j).astype(jnp.float32), axis=1,
                             keepdims=True)
                row = m_ref[:, pl.ds(j, 1), :][:, 0, :]
                sup = row * kj * (lane > j).astype(jnp.float32)
                return kb * (1.0 - sup)

            kb = lax.fori_loop(0, BLK, body, keep_ref[:, sl])
            keep_ref[:, sl] = kb

            for c in range(b + 1, NBLK):
                slc = slice(c * BLK, (c + 1) * BLK)
                m = _pair_iou_mask(bx1, by1, bx2, by2, br,
                                   x1[:, slc], y1[:, slc], x2[:, slc],
                                   y2[:, slc], area[:, slc])
                hit = jnp.max(m * kb[:, :, None], axis=1)
                keep_ref[:, slc] = keep_ref[:, slc] * (1.0 - hit)

        if (b + 1) * BLK >= OUT_PAD and b < NBLK - 1:
            cnt = jnp.sum(keep_ref[:, :(b + 1) * BLK], axis=1)
            done = jnp.logical_or(done, jnp.min(cnt) >= OUT_PAD)

    keep = keep_ref[...]
    csum = keep
    for sh in (1, 2, 4, 8, 16, 32, 64, 128, 256, 512, 1024):
        csum = csum + jnp.concatenate(
            [jnp.zeros((B, sh), jnp.float32), csum[:, :T - sh]], axis=1)
    pos = csum - 1.0

    zeros = jnp.zeros_like(score)
    data = jnp.stack([x1, y1, x2, y2, score, zeros, zeros, zeros], axis=1)
    data = data * keep[:, None, :]

    for o in range(OUT_PAD // BLK):
        tgt = (jnp.float32(o * BLK)
               + lax.broadcasted_iota(jnp.int32, (1, 1, BLK), 2).astype(jnp.float32))
        onehot = (pos[:, :, None] == tgt).astype(jnp.float32) * keep[:, :, None]
        out_ref[:, :, o * BLK:(o + 1) * BLK] = lax.dot_general(
            data, onehot,
            dimension_numbers=(((2,), (1,)), ((0,), (0,))),
            preferred_element_type=jnp.float32)


def _run_nms(scr, deltas_t, anc_t):
    B = scr.shape[0]
    return pl.pallas_call(
        _nms_kernel,
        out_shape=jax.ShapeDtypeStruct((B, 8, OUT_PAD), jnp.float32),
        scratch_shapes=[
            pltpu.VMEM((B, BLK, BLK), jnp.float32),
            pltpu.VMEM((B, T), jnp.float32),
        ],
    )(scr, deltas_t, anc_t)


_NW = 32
_BTOT = 8 * T
_BPW = _BTOT // _NW


def _sc_gather_body(*args):
    idx_hbm = args[0:8]
    dtab_hbm, atab_hbm = args[8], args[9]
    out_hbm = args[10:18]
    idx_v = args[18:26]
    row_v = args[26:34]
    sems = args[34:42]
    wid = lax.axis_index("s") * 2 + lax.axis_index("c")
    base = wid * _BPW
    for c in range(8):
        pltpu.sync_copy(idx_hbm[c].at[pl.ds(base, _BPW)], idx_v[c])
    tabs = [dtab_hbm] * 4 + [atab_hbm] * 4
    cps = [pltpu.async_copy(tabs[c].at[idx_v[c]], row_v[c], sems[c])
           for c in range(8)]
    for c in range(8):
        cps[c].wait()
        pltpu.sync_copy(row_v[c], out_hbm[c].at[pl.ds(base, _BPW)])


def _sc_gather(idx8, dtab, atab):
    return pl.kernel(
        _sc_gather_body,
        out_type=[jax.ShapeDtypeStruct((_BTOT,), jnp.float32)] * 8,
        mesh=plsc.VectorSubcoreMesh(core_axis_name="c", subcore_axis_name="s"),
        compiler_params=pltpu.CompilerParams(use_tc_tiling_on_sc=False),
        scratch_types=(
            [pltpu.VMEM((_BPW,), jnp.int32)] * 8
            + [pltpu.VMEM((_BPW,), jnp.float32)] * 8
            + [pltpu.SemaphoreType.DMA] * 8
        ),
    )(*idx8, dtab, atab)


def kernel(objectness, pred_bbox_deltas, anchors):
    B, N = objectness.shape
    top_scores, top_idx = lax.top_k(objectness, PRE_NMS_TOP_N)

    aidx = jnp.pad(top_idx, ((0, 0), (0, T - PRE_NMS_TOP_N))).reshape(-1)
    didx = (jnp.pad(top_idx, ((0, 0), (0, T - PRE_NMS_TOP_N)))
            + (jnp.arange(B, dtype=top_idx.dtype) * N)[:, None]).reshape(-1)
    idx8 = [didx * 4 + c for c in range(4)] + [aidx * 4 + c for c in range(4)]
    planar = _sc_gather(idx8, pred_bbox_deltas.reshape(-1),
                        anchors.reshape(-1))
    deltas_t = jnp.stack([p.reshape(B, T)[:, :PRE_NMS_TOP_N]
                          for p in planar[0:4]], axis=1)
    anc_t = jnp.stack([p.reshape(B, T)[:, :PRE_NMS_TOP_N]
                       for p in planar[4:8]], axis=1)

    out = _run_nms(top_scores, deltas_t, anc_t)
    out = jnp.transpose(out, (0, 2, 1))[:, :POST_NMS_TOP_N, :5]
    return out
